# baseline (speedup 1.0000x reference)
import functools

import jax
import jax.numpy as jnp
from jax import lax
from jax.experimental import pallas as pl
from jax.experimental.pallas import tpu as pltpu
from jax.experimental.pallas import tpu_sc as plsc

_NC = 2
_NS = 16
_LN = 16
_NW = _NC * _NS


def _table_body(emb_ref, w_ref, out_ref):
    logits = jnp.dot(emb_ref[...], w_ref[...],
                     preferred_element_type=jnp.float32)
    m = jnp.max(logits, axis=1, keepdims=True)
    lse = m + jnp.log(jnp.sum(jnp.exp(logits - m), axis=1, keepdims=True))
    nll = lse - logits
    padded = jnp.pad(nll, ((0, 0), (0, 1024 - nll.shape[1])))
    out_ref[...] = padded.reshape(-1)


def _build_nll_table(emb, w_out):
    v, d = emb.shape
    blk = 200
    steps = v // blk
    return pl.pallas_call(
        _table_body,
        grid=(steps,),
        in_specs=[
            pl.BlockSpec((blk, d), lambda i: (i, 0)),
            pl.BlockSpec((d, v), lambda i: (0, 0)),
        ],
        out_specs=pl.BlockSpec((blk * 1024,), lambda i: (i,)),
        out_shape=jax.ShapeDtypeStruct((v * 1024,), jnp.float32),
    )(emb, w_out)


def _sc_gather_partials(table_flat, xf, vocab, s_sz):
    total = xf.shape[0]
    c = total // _NW
    gw = 128
    ng = c // gw
    mesh = plsc.VectorSubcoreMesh(
        core_axis_name="c", subcore_axis_name="s",
        num_cores=_NC, num_subcores=_NS)

    @functools.partial(
        pl.kernel,
        out_type=jax.ShapeDtypeStruct((_NW, _LN), jnp.float32),
        mesh=mesh,
        scratch_types=[
            pltpu.VMEM((c + _LN,), jnp.int32),
            pltpu.VMEM((ng, gw), jnp.int32),
            pltpu.VMEM((ng, gw), jnp.float32),
            pltpu.VMEM((_LN,), jnp.float32),
            pltpu.SemaphoreType.DMA,
        ],
    )
    def k(table_hbm, x_hbm, out_hbm, xv, idx_v, val_v, vec_v, sem):
        cid = lax.axis_index("c")
        sid = lax.axis_index("s")
        wid = sid * _NC + cid
        base = wid * c

        @pl.when(wid == _NW - 1)
        def _():
            pltpu.sync_copy(x_hbm.at[pl.ds(base, c)], xv.at[pl.ds(0, c)])

        @pl.when(wid != _NW - 1)
        def _():
            pltpu.sync_copy(x_hbm.at[pl.ds(base, c + 8)],
                            xv.at[pl.ds(0, c + 8)])

        lanes = lax.iota(jnp.int32, _LN)
        shift = jnp.where(lanes == _LN - 1, 0, lanes + 1)
        zeros = jnp.zeros((_LN,), jnp.int32)
        last = lanes == _LN - 1
        for j in range(ng):
            for i in range(gw // _LN):
                o = j * gw + i * _LN
                t = xv[pl.ds(o, _LN)]
                t_next = xv[pl.ds(o + _LN, _LN)]
                lbl = jnp.where(
                    last,
                    t_next.at[zeros].get(mode='promise_in_bounds'),
                    t.at[shift].get(mode='promise_in_bounds'))
                p = base + o + lanes
                valid = lax.rem(p, s_sz) != s_sz - 1
                idx_v[j, pl.ds(i * _LN, _LN)] = jnp.where(
                    valid, t * 1024 + lbl, 0)
        copies = [pltpu.async_copy(table_hbm.at[idx_v.at[j]], val_v.at[j],
                                   sem) for j in range(ng)]
        for cp in copies:
            cp.wait()
        acc = jnp.zeros((_LN,), jnp.float32)
        for j in range(ng):
            for i in range(gw // _LN):
                p = base + j * gw + i * _LN + lanes
                vals = val_v[j, pl.ds(i * _LN, _LN)]
                acc = acc + jnp.where(lax.rem(p, s_sz) != s_sz - 1,
                                      vals, 0.0)
        vec_v[...] = acc
        pltpu.sync_copy(vec_v, out_hbm.at[wid])

    return k(table_flat, xf)


def kernel(x, emb, w_out):
    vocab = emb.shape[0]
    b, s = x.shape
    n_valid = b * (s - 1)
    table = _build_nll_table(emb, w_out)
    partials = _sc_gather_partials(table,
                                   x.astype(jnp.int32).reshape(-1), vocab, s)
    return jnp.sum(partials) / n_valid


# speedup vs baseline: 1.0242x; 1.0242x over previous
import functools

import jax
import jax.numpy as jnp
from jax import lax
from jax.experimental import pallas as pl
from jax.experimental.pallas import tpu as pltpu
from jax.experimental.pallas import tpu_sc as plsc

_NC = 2
_NS = 16
_LN = 16
_NW = _NC * _NS


_BLK = 200


def _table_body(emb_hbm, w_hbm, out_hbm, w_v, emb_v, out_v,
                sem_w, sem_in, sem_out):
    v = w_v.shape[1]
    steps = v // _BLK
    row_sz = _BLK * 1024

    def in_cp(i, slot):
        return pltpu.make_async_copy(
            emb_hbm.at[pl.ds(i * _BLK, _BLK), :], emb_v.at[slot], sem_in)

    def out_cp(i, slot):
        return pltpu.make_async_copy(
            out_v.at[slot], out_hbm.at[pl.ds(i * row_sz, row_sz)], sem_out)

    w_cp = pltpu.make_async_copy(w_hbm, w_v, sem_w)
    w_cp.start()
    in_cp(0, 0).start()
    for i in range(steps):
        if i + 1 < steps:
            in_cp(i + 1, (i + 1) % 2).start()
        in_cp(i, i % 2).wait()
        if i == 0:
            w_cp.wait()
        logits = jnp.dot(emb_v[i % 2], w_v[...],
                         preferred_element_type=jnp.float32)
        m = jnp.max(logits, axis=1, keepdims=True)
        lse = m + jnp.log(jnp.sum(jnp.exp(logits - m), axis=1,
                                  keepdims=True))
        nll = lse - logits
        padded = jnp.pad(nll, ((0, 0), (0, 1024 - nll.shape[1])))
        if i >= 2:
            out_cp(i - 2, i % 2).wait()
        out_v[i % 2] = padded.reshape(-1)
        out_cp(i, i % 2).start()
    for i in range(max(steps - 2, 0), steps):
        out_cp(i, i % 2).wait()


def _build_nll_table(emb, w_out):
    v, d = emb.shape
    return pl.pallas_call(
        _table_body,
        in_specs=[
            pl.BlockSpec(memory_space=pl.ANY),
            pl.BlockSpec(memory_space=pl.ANY),
        ],
        out_specs=pl.BlockSpec(memory_space=pl.ANY),
        out_shape=jax.ShapeDtypeStruct((v * 1024,), jnp.float32),
        scratch_shapes=[
            pltpu.VMEM((d, v), jnp.float32),
            pltpu.VMEM((2, _BLK, d), jnp.float32),
            pltpu.VMEM((2, _BLK * 1024), jnp.float32),
            pltpu.SemaphoreType.DMA,
            pltpu.SemaphoreType.DMA,
            pltpu.SemaphoreType.DMA,
        ],
    )(emb, w_out)


def _sc_gather_partials(table_flat, xf, vocab, s_sz):
    total = xf.shape[0]
    c = total // _NW
    gw = 128
    ng = c // gw
    mesh = plsc.VectorSubcoreMesh(
        core_axis_name="c", subcore_axis_name="s",
        num_cores=_NC, num_subcores=_NS)

    @functools.partial(
        pl.kernel,
        out_type=jax.ShapeDtypeStruct((_NW, _LN), jnp.float32),
        mesh=mesh,
        scratch_types=[
            pltpu.VMEM((c + _LN,), jnp.int32),
            pltpu.VMEM((ng, gw), jnp.int32),
            pltpu.VMEM((ng, gw), jnp.float32),
            pltpu.VMEM((_LN,), jnp.float32),
            pltpu.SemaphoreType.DMA,
        ],
    )
    def k(table_hbm, x_hbm, out_hbm, xv, idx_v, val_v, vec_v, sem):
        cid = lax.axis_index("c")
        sid = lax.axis_index("s")
        wid = sid * _NC + cid
        base = wid * c

        @pl.when(wid == _NW - 1)
        def _():
            pltpu.sync_copy(x_hbm.at[pl.ds(base, c)], xv.at[pl.ds(0, c)])

        @pl.when(wid != _NW - 1)
        def _():
            pltpu.sync_copy(x_hbm.at[pl.ds(base, c + 8)],
                            xv.at[pl.ds(0, c + 8)])

        lanes = lax.iota(jnp.int32, _LN)
        shift = jnp.where(lanes == _LN - 1, 0, lanes + 1)
        zeros = jnp.zeros((_LN,), jnp.int32)
        last = lanes == _LN - 1
        for j in range(ng):
            for i in range(gw // _LN):
                o = j * gw + i * _LN
                t = xv[pl.ds(o, _LN)]
                t_next = xv[pl.ds(o + _LN, _LN)]
                lbl = jnp.where(
                    last,
                    t_next.at[zeros].get(mode='promise_in_bounds'),
                    t.at[shift].get(mode='promise_in_bounds'))
                p = base + o + lanes
                valid = lax.rem(p, s_sz) != s_sz - 1
                idx_v[j, pl.ds(i * _LN, _LN)] = jnp.where(
                    valid, t * 1024 + lbl, 0)
        copies = [pltpu.async_copy(table_hbm.at[idx_v.at[j]], val_v.at[j],
                                   sem) for j in range(ng)]
        for cp in copies:
            cp.wait()
        acc = jnp.zeros((_LN,), jnp.float32)
        for j in range(ng):
            for i in range(gw // _LN):
                p = base + j * gw + i * _LN + lanes
                vals = val_v[j, pl.ds(i * _LN, _LN)]
                acc = acc + jnp.where(lax.rem(p, s_sz) != s_sz - 1,
                                      vals, 0.0)
        vec_v[...] = acc
        pltpu.sync_copy(vec_v, out_hbm.at[wid])

    return k(table_flat, xf)


def kernel(x, emb, w_out):
    vocab = emb.shape[0]
    b, s = x.shape
    n_valid = b * (s - 1)
    table = _build_nll_table(emb, w_out)
    partials = _sc_gather_partials(table,
                                   x.astype(jnp.int32).reshape(-1), vocab, s)
    return jnp.sum(partials) / n_valid
